# baseline (speedup 1.0000x reference)
import dataclasses

import jax
import jax.numpy as jnp
from jax import lax
from jax.experimental import pallas as pl
from jax.experimental.pallas import tpu as pltpu
from jax.experimental.pallas import tpu_sc as plsc

B, L = 1024, 200
N = B * L
WORD_D = 64
AUX_D = 32
AUXS_D = 4 * AUX_D
OUT_D = WORD_D + AUXS_D

POS_V, NER_V, DEPREL_V = 56, 24, 48
MAX_SRC = 200
AUX_BASE = (0, POS_V, POS_V + NER_V, POS_V + NER_V + DEPREL_V)
AUX_ROWS = POS_V + NER_V + DEPREL_V + MAX_SRC + 1
AUX_ROWS_PAD = 336

NUM_CORES = 2
NUM_SUBCORES = 16
NW = NUM_CORES * NUM_SUBCORES
PER_W = N // NW
CHUNK = 128
NCHUNK = PER_W // CHUNK

LANES = 16


def _emb_kernel(word_hbm, auxtab_hbm, idx_hbm, out_hbm,
                idx_v, aux_t, wbuf0, wbuf1, abuf0, abuf1, gsem, wsem):
    wid = lax.axis_index("s") * NUM_CORES + lax.axis_index("c")
    crow = wid * NCHUNK

    pltpu.sync_copy(idx_hbm.at[:, pl.ds(crow, NCHUNK)], idx_v)
    pltpu.sync_copy(auxtab_hbm, aux_t)

    wbufs = (wbuf0, wbuf1)
    abufs = (abuf0, abuf1)

    io0 = lax.iota(jnp.int32, LANES)
    io1 = io0 + LANES
    ktabs = [jnp.full((LANES,), k + 1, jnp.int32) for k in range(4)]

    def aux_fill(i, abuf):
        vi = jnp.full((LANES,), i, jnp.int32)

        @pl.loop(0, CHUNK)
        def _(t):
            vt = jnp.full((LANES,), t, jnp.int32)
            for k in range(4):
                v = plsc.load_gather(idx_v, [ktabs[k], vi, vt]) + AUX_BASE[k]
                lo = plsc.load_gather(aux_t, [v, io0])
                hi = plsc.load_gather(aux_t, [v, io1])
                abuf[t, pl.ds(k * AUX_D, LANES)] = lo
                abuf[t, pl.ds(k * AUX_D + LANES, LANES)] = hi

    def wb_drain(s):
        pltpu.make_async_copy(
            wbufs[s], out_hbm.at[pl.ds(0, CHUNK), pl.ds(0, WORD_D)],
            wsem).wait()
        pltpu.make_async_copy(
            abufs[s], out_hbm.at[pl.ds(0, CHUNK), pl.ds(WORD_D, AUXS_D)],
            wsem).wait()

    def do_chunk(i, s):
        gh = pltpu.async_copy(
            word_hbm.at[idx_v.at[0, i]], wbufs[s], gsem)
        aux_fill(i, abufs[s])
        gh.wait()
        row0 = (crow + i) * CHUNK
        pltpu.async_copy(
            wbufs[s], out_hbm.at[pl.ds(row0, CHUNK), pl.ds(0, WORD_D)], wsem)
        pltpu.async_copy(
            abufs[s], out_hbm.at[pl.ds(row0, CHUNK), pl.ds(WORD_D, AUXS_D)],
            wsem)

    @pl.loop(0, NCHUNK // 2)
    def _(m):
        for s in (0, 1):
            @pl.when(m > 0)
            def _():
                wb_drain(s)
            do_chunk(2 * m + s, s)

    wb_drain(0)
    wb_drain(1)


def _compiler_params():
    cp = pltpu.CompilerParams(use_tc_tiling_on_sc=False)
    if "needs_layout_passes" in pltpu.CompilerParams.__dataclass_fields__:
        cp = dataclasses.replace(cp, needs_layout_passes=False)
    return cp


@jax.jit
def kernel(word_table, pos_table, ner_table, deprel_table, position_table,
           word_rep, pos_rep, ner_rep, deprel_rep, position_rep):
    aux_tab = jnp.concatenate(
        [pos_table, ner_table, deprel_table, position_table,
         jnp.zeros((AUX_ROWS_PAD - AUX_ROWS, AUX_D), jnp.float32)], axis=0)
    idx = jnp.stack(
        [word_rep.reshape(N // CHUNK, CHUNK).astype(jnp.int32),
         pos_rep.reshape(N // CHUNK, CHUNK).astype(jnp.int32),
         ner_rep.reshape(N // CHUNK, CHUNK).astype(jnp.int32),
         deprel_rep.reshape(N // CHUNK, CHUNK).astype(jnp.int32),
         position_rep.reshape(N // CHUNK, CHUNK).astype(jnp.int32)], axis=0)

    mesh = plsc.VectorSubcoreMesh(core_axis_name="c", subcore_axis_name="s")
    run = pl.kernel(
        _emb_kernel,
        out_type=jax.ShapeDtypeStruct((N, OUT_D), jnp.float32),
        mesh=mesh,
        compiler_params=_compiler_params(),
        scratch_types=(
            [pltpu.VMEM((5, NCHUNK, CHUNK), jnp.int32),
             pltpu.VMEM((AUX_ROWS_PAD, AUX_D), jnp.float32)]
            + [pltpu.VMEM((CHUNK, WORD_D), jnp.float32) for _ in range(2)]
            + [pltpu.VMEM((CHUNK, AUXS_D), jnp.float32) for _ in range(2)]
            + [pltpu.SemaphoreType.DMA] * 2
        ),
    )
    out = run(word_table, aux_tab, idx)
    return out.reshape(B, L, OUT_D)


# speedup vs baseline: 1.1664x; 1.1664x over previous
import dataclasses

import jax
import jax.numpy as jnp
from jax import lax
from jax.experimental import pallas as pl
from jax.experimental.pallas import tpu as pltpu
from jax.experimental.pallas import tpu_sc as plsc

B, L = 1024, 200
N = B * L
WORD_D = 64
AUX_D = 32
AUXS_D = 4 * AUX_D
OUT_D = WORD_D + AUXS_D

POS_V, NER_V, DEPREL_V = 56, 24, 48
MAX_SRC = 200
AUX_BASE = (0, POS_V, POS_V + NER_V, POS_V + NER_V + DEPREL_V)
AUX_ROWS = POS_V + NER_V + DEPREL_V + MAX_SRC + 1
AUX_ROWS_PAD = 336

NUM_CORES = 2
NUM_SUBCORES = 16
NW = NUM_CORES * NUM_SUBCORES
CHUNK = 128
NBB = B // CHUNK
NLQ = NW // NBB
LQ = L // NLQ
NCHUNK = LQ

LANES = 16


def _emb_kernel(word_hbm, auxtab_hbm,
                widx_hbm, pidx_hbm, nidx_hbm, didx_hbm, xidx_hbm,
                out_hbm,
                idx_v, aux_t, wbuf0, wbuf1, abuf0, abuf1, gsem, wsem):
    wid = lax.axis_index("s") * NUM_CORES + lax.axis_index("c")
    lq = wid // NBB
    cq = wid - lq * NBB
    l0 = lq * LQ
    b0 = cq * CHUNK

    pltpu.sync_copy(widx_hbm.at[pl.ds(l0, LQ), pl.ds(b0, CHUNK)],
                    idx_v.at[0])
    pltpu.sync_copy(pidx_hbm.at[pl.ds(l0, LQ), pl.ds(b0, CHUNK)],
                    idx_v.at[1])
    pltpu.sync_copy(nidx_hbm.at[pl.ds(l0, LQ), pl.ds(b0, CHUNK)],
                    idx_v.at[2])
    pltpu.sync_copy(didx_hbm.at[pl.ds(l0, LQ), pl.ds(b0, CHUNK)],
                    idx_v.at[3])
    pltpu.sync_copy(xidx_hbm.at[pl.ds(l0, LQ), pl.ds(b0, CHUNK)],
                    idx_v.at[4])
    pltpu.sync_copy(auxtab_hbm, aux_t)

    wbufs = (wbuf0, wbuf1)
    abufs = (abuf0, abuf1)

    io0 = lax.iota(jnp.int32, LANES)
    io1 = io0 + LANES

    def aux_fill(i, abuf):
        @pl.loop(0, CHUNK // LANES)
        def _(g):
            t0 = g * LANES
            for k in range(4):
                vec = idx_v[k + 1, i, pl.ds(t0, LANES)] + AUX_BASE[k]
                for j in range(LANES):
                    v = jnp.broadcast_to(vec[j], (LANES,))
                    lo = plsc.load_gather(aux_t, [v, io0])
                    hi = plsc.load_gather(aux_t, [v, io1])
                    abuf[t0 + j, pl.ds(k * AUX_D, LANES)] = lo
                    abuf[t0 + j, pl.ds(k * AUX_D + LANES, LANES)] = hi

    def wb_drain(s):
        pltpu.make_async_copy(
            wbufs[s], out_hbm.at[pl.ds(b0, CHUNK), l0, pl.ds(0, WORD_D)],
            wsem).wait()
        pltpu.make_async_copy(
            abufs[s], out_hbm.at[pl.ds(b0, CHUNK), l0, pl.ds(WORD_D, AUXS_D)],
            wsem).wait()

    def do_chunk(i, s):
        gh = pltpu.async_copy(
            word_hbm.at[idx_v.at[0, i]], wbufs[s], gsem)
        aux_fill(i, abufs[s])
        gh.wait()
        l = l0 + i
        pltpu.async_copy(
            wbufs[s], out_hbm.at[pl.ds(b0, CHUNK), l, pl.ds(0, WORD_D)],
            wsem)
        pltpu.async_copy(
            abufs[s], out_hbm.at[pl.ds(b0, CHUNK), l, pl.ds(WORD_D, AUXS_D)],
            wsem)

    @pl.loop(0, NCHUNK // 2)
    def _(m):
        for s in (0, 1):
            @pl.when(m > 0)
            def _():
                wb_drain(s)
            do_chunk(2 * m + s, s)

    wb_drain(0)
    wb_drain(1)


def _compiler_params():
    cp = pltpu.CompilerParams(use_tc_tiling_on_sc=False)
    if "needs_layout_passes" in pltpu.CompilerParams.__dataclass_fields__:
        cp = dataclasses.replace(cp, needs_layout_passes=False)
    return cp


@jax.jit
def kernel(word_table, pos_table, ner_table, deprel_table, position_table,
           word_rep, pos_rep, ner_rep, deprel_rep, position_rep):
    aux_tab = jnp.concatenate(
        [pos_table, ner_table, deprel_table, position_table,
         jnp.zeros((AUX_ROWS_PAD - AUX_ROWS, AUX_D), jnp.float32)], axis=0)

    mesh = plsc.VectorSubcoreMesh(core_axis_name="c", subcore_axis_name="s")
    run = pl.kernel(
        _emb_kernel,
        out_type=jax.ShapeDtypeStruct((B, L, OUT_D), jnp.float32),
        mesh=mesh,
        compiler_params=_compiler_params(),
        scratch_types=(
            [pltpu.VMEM((5, NCHUNK, CHUNK), jnp.int32),
             pltpu.VMEM((AUX_ROWS_PAD, AUX_D), jnp.float32)]
            + [pltpu.VMEM((CHUNK, WORD_D), jnp.float32) for _ in range(2)]
            + [pltpu.VMEM((CHUNK, AUXS_D), jnp.float32) for _ in range(2)]
            + [pltpu.SemaphoreType.DMA] * 2
        ),
    )
    out = run(word_table, aux_tab,
              word_rep.T.astype(jnp.int32), pos_rep.T.astype(jnp.int32),
              ner_rep.T.astype(jnp.int32), deprel_rep.T.astype(jnp.int32),
              position_rep.T.astype(jnp.int32))
    return out


# speedup vs baseline: 1.1721x; 1.0049x over previous
import dataclasses

import jax
import jax.numpy as jnp
from jax import lax
from jax.experimental import pallas as pl
from jax.experimental.pallas import tpu as pltpu
from jax.experimental.pallas import tpu_sc as plsc

B, L = 1024, 200
N = B * L
WORD_D = 64
AUX_D = 32
AUXS_D = 4 * AUX_D
OUT_D = WORD_D + AUXS_D

POS_V, NER_V, DEPREL_V = 56, 24, 48
MAX_SRC = 200
AUX_BASE = (0, POS_V, POS_V + NER_V, POS_V + NER_V + DEPREL_V)
AUX_ROWS = POS_V + NER_V + DEPREL_V + MAX_SRC + 1
AUX_ROWS_PAD = 336

NUM_CORES = 2
NUM_SUBCORES = 16
NW = NUM_CORES * NUM_SUBCORES
PER_W = N // NW
CHUNK = 128
NCHUNK = PER_W // CHUNK

LANES = 16


def _emb_kernel(word_hbm, auxtab_hbm, idx_hbm, out_hbm,
                idx_v, aux_t, wbuf0, wbuf1, abuf0, abuf1, gsem, wsem):
    wid = lax.axis_index("s") * NUM_CORES + lax.axis_index("c")
    crow = wid * NCHUNK

    pltpu.sync_copy(idx_hbm.at[:, pl.ds(crow, NCHUNK)], idx_v)
    pltpu.sync_copy(auxtab_hbm, aux_t)

    wbufs = (wbuf0, wbuf1)
    abufs = (abuf0, abuf1)

    io0 = lax.iota(jnp.int32, LANES)
    io1 = io0 + LANES

    def aux_fill(i, abuf):
        @pl.loop(0, CHUNK // LANES)
        def _(g):
            t0 = g * LANES
            for k in range(4):
                vec = idx_v[k + 1, i, pl.ds(t0, LANES)] + AUX_BASE[k]
                for j in range(LANES):
                    v = jnp.broadcast_to(vec[j], (LANES,))
                    lo = plsc.load_gather(aux_t, [v, io0])
                    hi = plsc.load_gather(aux_t, [v, io1])
                    abuf[t0 + j, pl.ds(k * AUX_D, LANES)] = lo
                    abuf[t0 + j, pl.ds(k * AUX_D + LANES, LANES)] = hi

    def wb_drain(s):
        pltpu.make_async_copy(
            wbufs[s], out_hbm.at[pl.ds(0, CHUNK), pl.ds(0, WORD_D)],
            wsem).wait()
        pltpu.make_async_copy(
            abufs[s], out_hbm.at[pl.ds(0, CHUNK), pl.ds(WORD_D, AUXS_D)],
            wsem).wait()

    def do_chunk(i, s):
        gh = pltpu.async_copy(
            word_hbm.at[idx_v.at[0, i]], wbufs[s], gsem)
        aux_fill(i, abufs[s])
        gh.wait()
        row0 = (crow + i) * CHUNK
        pltpu.async_copy(
            wbufs[s], out_hbm.at[pl.ds(row0, CHUNK), pl.ds(0, WORD_D)], wsem)
        pltpu.async_copy(
            abufs[s], out_hbm.at[pl.ds(row0, CHUNK), pl.ds(WORD_D, AUXS_D)],
            wsem)

    @pl.loop(0, NCHUNK // 2)
    def _(m):
        for s in (0, 1):
            @pl.when(m > 0)
            def _():
                wb_drain(s)
            do_chunk(2 * m + s, s)

    wb_drain(0)
    wb_drain(1)


def _compiler_params():
    cp = pltpu.CompilerParams(use_tc_tiling_on_sc=False)
    if "needs_layout_passes" in pltpu.CompilerParams.__dataclass_fields__:
        cp = dataclasses.replace(cp, needs_layout_passes=False)
    return cp


@jax.jit
def kernel(word_table, pos_table, ner_table, deprel_table, position_table,
           word_rep, pos_rep, ner_rep, deprel_rep, position_rep):
    aux_tab = jnp.concatenate(
        [pos_table, ner_table, deprel_table, position_table,
         jnp.zeros((AUX_ROWS_PAD - AUX_ROWS, AUX_D), jnp.float32)], axis=0)
    idx = jnp.stack(
        [word_rep.reshape(N // CHUNK, CHUNK).astype(jnp.int32),
         pos_rep.reshape(N // CHUNK, CHUNK).astype(jnp.int32),
         ner_rep.reshape(N // CHUNK, CHUNK).astype(jnp.int32),
         deprel_rep.reshape(N // CHUNK, CHUNK).astype(jnp.int32),
         position_rep.reshape(N // CHUNK, CHUNK).astype(jnp.int32)], axis=0)

    mesh = plsc.VectorSubcoreMesh(core_axis_name="c", subcore_axis_name="s")
    run = pl.kernel(
        _emb_kernel,
        out_type=jax.ShapeDtypeStruct((N, OUT_D), jnp.float32),
        mesh=mesh,
        compiler_params=_compiler_params(),
        scratch_types=(
            [pltpu.VMEM((5, NCHUNK, CHUNK), jnp.int32),
             pltpu.VMEM((AUX_ROWS_PAD, AUX_D), jnp.float32)]
            + [pltpu.VMEM((CHUNK, WORD_D), jnp.float32) for _ in range(2)]
            + [pltpu.VMEM((CHUNK, AUXS_D), jnp.float32) for _ in range(2)]
            + [pltpu.SemaphoreType.DMA] * 2
        ),
    )
    out = run(word_table, aux_tab, idx)
    return out.reshape(B, L, OUT_D)
